# baseline (speedup 1.0000x reference)
import functools

import jax
import jax.numpy as jnp
from jax import lax
from jax.experimental import pallas as pl
from jax.experimental.pallas import tpu as pltpu
from jax.experimental.pallas import tpu_sc as plsc

N = 10000
E = 320000
D = 128
H = 128
TOPO = 32
C = 10
G = 64

NC = 2
NS = 16
NW = NC * NS
NPAD = 10240
ROWS_PER_SUB = NPAD // NS
CHUNK = 128
EPT = 10240
EPAD = EPT * NW
NCHUNK = EPT // CHUNK
DEGW = 16
NTAB = NPAD


def _sc_mesh():
    return plsc.VectorSubcoreMesh(
        core_axis_name="c", subcore_axis_name="s", num_cores=NC, num_subcores=NS
    )


EPC = NS * EPT
DCH = 2048
NDCH = EPC // DCH
SLC = NPAD // NS


@functools.partial(
    pl.kernel,
    out_type=jax.ShapeDtypeStruct((NC, NPAD, 16), jnp.float32),
    mesh=_sc_mesh(),
    scratch_types=[
        pltpu.VMEM((2, DCH), jnp.int32),
        pltpu.VMEM((SLC, 16), jnp.float32),
        pltpu.SemaphoreType.DMA((2,)),
    ],
    compiler_params=pltpu.CompilerParams(needs_layout_passes=False),
)
def _sc_degree(dst_hbm, out_hbm, dbuf, hist, isem):
    cid = lax.axis_index("c")
    sid = lax.axis_index("s")
    lo = sid * SLC

    zeros16 = jnp.zeros((16,), jnp.float32)

    def zbody(v, carry):
        hist[v, :] = zeros16
        return carry

    lax.fori_loop(0, SLC, zbody, 0)

    lane = lax.iota(jnp.int32, 16)

    def dload(i, b):
        pltpu.async_copy(dst_hbm.at[cid, pl.ds(i * DCH, DCH)], dbuf.at[b],
                         isem.at[b])

    def dload_wait(i, b):
        pltpu.make_async_copy(dst_hbm.at[cid, pl.ds(i * DCH, DCH)], dbuf.at[b],
                              isem.at[b]).wait()

    dload(0, 0)

    def inner(b):
        def gbody(g, carry):
            idx16 = dbuf[b, pl.ds(g * 16, 16)]
            rl = idx16 - lo
            m = (rl >= 0) & (rl < SLC)
            rls = jnp.where(m, rl, 0)
            cur = plsc.load_gather(hist, [rls, lane], mask=m)
            plsc.store_scatter(hist, [rls, lane], cur + 1.0, mask=m)
            return carry

        lax.fori_loop(0, DCH // 16, gbody, 0)

    def body(gr, carry):
        for b in range(2):
            i = gr * 2 + b

            @pl.when(i + 1 < NDCH)
            def _():
                dload(i + 1, 1 - b)

            dload_wait(i, b)
            inner(b)
        return carry

    lax.fori_loop(0, NDCH // 2, body, 0)

    pltpu.sync_copy(hist, out_hbm.at[cid, pl.ds(lo, SLC)])


@functools.partial(
    pl.kernel,
    out_type=jax.ShapeDtypeStruct((NC, NPAD, H), jnp.float32),
    mesh=_sc_mesh(),
    scratch_types=[
        pltpu.VMEM((2, CHUNK), jnp.int32),
        pltpu.VMEM((NCHUNK, CHUNK), jnp.int32),
        pltpu.VMEM((2, CHUNK, H), jnp.float32),
        pltpu.SemaphoreType.DMA((2,)),
        pltpu.SemaphoreType.DMA((2,)),
        pltpu.SemaphoreType.DMA((2,)),
        pltpu.VMEM_SHARED((NPAD, H), jnp.float32),
    ],
)
def _sc_spmm(table_hbm, src_hbm, dst_hbm, zrow_hbm, out_hbm,
             si_v, dst_all, rows_v, isem, gsem, ssem, acc_sh):
    cid = lax.axis_index("c")
    sid = lax.axis_index("s")
    wid = cid * NS + sid

    pltpu.sync_copy(dst_hbm.at[wid], dst_all)
    pltpu.sync_copy(zrow_hbm, acc_sh.at[pl.ds(sid * ROWS_PER_SUB, ROWS_PER_SUB)])
    plsc.subcore_barrier()

    def idxload(i, b):
        pltpu.async_copy(src_hbm.at[wid, i], si_v.at[b], isem.at[b])

    def idx_wait(i, b):
        pltpu.make_async_copy(src_hbm.at[wid, i], si_v.at[b], isem.at[b]).wait()

    def gather(b):
        pltpu.async_copy(table_hbm.at[si_v.at[b]], rows_v.at[b], gsem.at[b])

    def gather_wait(b):
        pltpu.make_async_copy(
            table_hbm.at[si_v.at[b]], rows_v.at[b], gsem.at[b]
        ).wait()

    def scatter(i, b):
        pltpu.async_copy(rows_v.at[b], acc_sh.at[dst_all.at[i]], ssem.at[b],
                         add=True)

    def scatter_wait(i, b):
        pltpu.make_async_copy(
            rows_v.at[b], acc_sh.at[dst_all.at[i]], ssem.at[b]
        ).wait()

    idxload(0, 0)
    idxload(1, 1)
    idx_wait(0, 0)
    gather(0)

    def body(g, carry):
        for b in range(2):
            i2 = g * 2 + b
            nb = 1 - b

            @pl.when(i2 >= 1)
            def _():
                scatter_wait(i2 - 1, nb)

            @pl.when(i2 + 1 < NCHUNK)
            def _():
                idx_wait(i2 + 1, nb)
                gather(nb)

            gather_wait(b)
            scatter(i2, b)

            @pl.when(i2 + 2 < NCHUNK)
            def _():
                idxload(i2 + 2, b)
        return carry

    lax.fori_loop(0, NCHUNK // 2, body, 0)
    scatter_wait(NCHUNK - 1, (NCHUNK - 1) % 2)

    plsc.subcore_barrier()

    pltpu.sync_copy(
        acc_sh.at[pl.ds(sid * ROWS_PER_SUB, ROWS_PER_SUB)],
        out_hbm.at[cid, pl.ds(sid * ROWS_PER_SUB, ROWS_PER_SUB)],
    )


def _tc_prescale_body(x_ref, w_ref, degp_ref, s_ref, d2_ref, dinv_ref):
    deg = jnp.sum(degp_ref[0, 0:N, :] + degp_ref[1, 0:N, :], axis=1,
                  keepdims=True) + 1.0
    dinv = lax.rsqrt(deg)
    xw = jnp.dot(x_ref[...], w_ref[...], preferred_element_type=jnp.float32)
    s = dinv * xw
    s_ref[...] = s
    d2_ref[...] = dinv * s
    dinv_ref[...] = dinv


_tc_prescale = pl.pallas_call(
    _tc_prescale_body,
    out_shape=[
        jax.ShapeDtypeStruct((N, H), jnp.float32),
        jax.ShapeDtypeStruct((N, H), jnp.float32),
        jax.ShapeDtypeStruct((N, 1), jnp.float32),
    ],
)


def _tc_mid_body(p_ref, d2_ref, dinv_ref, b1_ref, w2_ref, s_ref, d2o_ref):
    agg = p_ref[0, 0:N, :] + p_ref[1, 0:N, :]
    dinv = dinv_ref[...]
    h1 = jnp.maximum(dinv * agg + d2_ref[...] + b1_ref[...], 0.0)
    xw = jnp.dot(h1, w2_ref[...], preferred_element_type=jnp.float32)
    s = dinv * xw
    s_ref[...] = s
    d2o_ref[...] = dinv * s


_tc_mid = pl.pallas_call(
    _tc_mid_body,
    out_shape=[
        jax.ShapeDtypeStruct((N, H), jnp.float32),
        jax.ShapeDtypeStruct((N, H), jnp.float32),
    ],
)


def _tc_head_body(q_ref, d2_ref, dinv_ref, b2_ref, batch_ref, topo_ref,
                  wl_ref, bl_ref, out_ref):
    agg = q_ref[0, 0:N, :] + q_ref[1, 0:N, :]
    h2 = jnp.maximum(dinv_ref[...] * agg + d2_ref[...] + b2_ref[...], 0.0)
    gid = lax.broadcasted_iota(jnp.int32, (N, G), 1)
    onehot = jnp.where(gid == batch_ref[...], 1.0, 0.0)
    ssum = lax.dot_general(
        onehot, h2, (((0,), (0,)), ((), ())), preferred_element_type=jnp.float32
    )
    cnt = lax.dot_general(
        onehot, jnp.ones((N, 1), jnp.float32), (((0,), (0,)), ((), ())),
        preferred_element_type=jnp.float32,
    )
    g = jnp.where(cnt > 0.0, ssum / jnp.maximum(cnt, 1.0), 0.0)
    gt = jnp.concatenate([g, topo_ref[...]], axis=1)
    out_ref[...] = (
        jnp.dot(gt, wl_ref[...], preferred_element_type=jnp.float32) + bl_ref[...]
    )


_tc_head = pl.pallas_call(
    _tc_head_body,
    out_shape=jax.ShapeDtypeStruct((G, C), jnp.float32),
)


def kernel(x, edge_index, batch, topo, W1, b1, W2, b2, Wl, bl):
    src = edge_index[0]
    dst = edge_index[1]
    pad = EPAD - E
    iota_pad = jnp.arange(pad, dtype=jnp.int32)
    src_p = jnp.concatenate([src, jnp.zeros((pad,), jnp.int32)])
    dst_msg = jnp.concatenate([dst, N + iota_pad % (NPAD - N)])
    dst_deg = jnp.concatenate([dst, N + iota_pad % (NPAD - N)])
    src3 = src_p.reshape(NW, NCHUNK, CHUNK)
    dst3m = dst_msg.reshape(NW, NCHUNK, CHUNK)
    dst2d = dst_deg.reshape(NC, NS * EPT)

    zrow = jnp.zeros((ROWS_PER_SUB, H), jnp.float32)

    degp = _sc_degree(dst2d)
    s1, d2xw1, dinv = _tc_prescale(x, W1, degp)
    p = _sc_spmm(s1, src3, dst3m, zrow)
    s2, d2xw2 = _tc_mid(p, d2xw1, dinv, b1.reshape(1, H), W2)
    q = _sc_spmm(s2, src3, dst3m, zrow)
    out = _tc_head(
        q, d2xw2, dinv, b2.reshape(1, H), batch.reshape(N, 1), topo,
        Wl, bl.reshape(1, C),
    )
    return out


# speedup vs baseline: 1.2197x; 1.2197x over previous
import functools

import jax
import jax.numpy as jnp
from jax import lax
from jax.experimental import pallas as pl
from jax.experimental.pallas import tpu as pltpu
from jax.experimental.pallas import tpu_sc as plsc

N = 10000
E = 320000
D = 128
H = 128
TOPO = 32
C = 10
G = 64

NC = 2
NS = 16
NW = NC * NS
NPAD = 10240
NACC = 10112
ROWS_PER_SUB = NACC // NS
CHUNK = 128
EPT = 10240
EPAD = EPT * NW
NCHUNK = EPT // CHUNK
NTAB = NACC

FAST_CORE = 0
CA = 128
CB = 32
CMAX = CA


def _sc_mesh():
    return plsc.VectorSubcoreMesh(
        core_axis_name="c", subcore_axis_name="s", num_cores=NC, num_subcores=NS
    )


EPC = NS * EPT
DCH = 2048
NDCH = EPC // DCH
SLC = NPAD // NS


@functools.partial(
    pl.kernel,
    out_type=jax.ShapeDtypeStruct((NC, NPAD, 16), jnp.float32),
    mesh=_sc_mesh(),
    scratch_types=[
        pltpu.VMEM((2, DCH), jnp.int32),
        pltpu.VMEM((SLC, 16), jnp.float32),
        pltpu.SemaphoreType.DMA((2,)),
    ],
    compiler_params=pltpu.CompilerParams(needs_layout_passes=False),
)
def _sc_degree(dst_hbm, out_hbm, dbuf, hist, isem):
    cid = lax.axis_index("c")
    sid = lax.axis_index("s")
    lo = sid * SLC

    zeros16 = jnp.zeros((16,), jnp.float32)

    def zbody(v, carry):
        hist[v, :] = zeros16
        return carry

    lax.fori_loop(0, SLC, zbody, 0)

    lane = lax.iota(jnp.int32, 16)

    def dload(i, b):
        pltpu.async_copy(dst_hbm.at[cid, pl.ds(i * DCH, DCH)], dbuf.at[b],
                         isem.at[b])

    def dload_wait(i, b):
        pltpu.make_async_copy(dst_hbm.at[cid, pl.ds(i * DCH, DCH)], dbuf.at[b],
                              isem.at[b]).wait()

    dload(0, 0)

    def inner(b):
        def gbody(g, carry):
            idx16 = dbuf[b, pl.ds(g * 16, 16)]
            rl = idx16 - lo
            m = (rl >= 0) & (rl < SLC)
            rls = jnp.where(m, rl, 0)
            cur = plsc.load_gather(hist, [rls, lane], mask=m)
            plsc.store_scatter(hist, [rls, lane], cur + 1.0, mask=m)
            return carry

        lax.fori_loop(0, DCH // 16, gbody, 0)

    def body(gr, carry):
        for b in range(2):
            i = gr * 2 + b

            @pl.when(i + 1 < NDCH)
            def _():
                dload(i + 1, 1 - b)

            dload_wait(i, b)
            inner(b)
        return carry

    lax.fori_loop(0, NDCH // 2, body, 0)

    pltpu.sync_copy(hist, out_hbm.at[cid, pl.ds(lo, SLC)])


@functools.partial(
    pl.kernel,
    out_type=jax.ShapeDtypeStruct((NC, NACC, H), jnp.float32),
    mesh=_sc_mesh(),
    scratch_types=[
        pltpu.VMEM((2, CHUNK), jnp.int32),
        pltpu.VMEM((CMAX, CHUNK), jnp.int32),
        pltpu.VMEM((2, CHUNK, H), jnp.float32),
        pltpu.SemaphoreType.DMA((2,)),
        pltpu.SemaphoreType.DMA((2,)),
        pltpu.SemaphoreType.DMA((2,)),
        pltpu.VMEM_SHARED((NACC, H), jnp.float32),
    ],
)
def _sc_spmm(table_hbm, src_hbm, dst_hbm, zrow_hbm, out_hbm,
             si_v, dst_all, rows_v, isem, gsem, ssem, acc_sh):
    cid = lax.axis_index("c")
    sid = lax.axis_index("s")
    wid = cid * NS + sid
    nch2 = jnp.where(cid == FAST_CORE, CA // 2, CB // 2)
    nch = nch2 * 2

    pltpu.sync_copy(dst_hbm.at[wid], dst_all)
    pltpu.sync_copy(zrow_hbm, acc_sh.at[pl.ds(sid * ROWS_PER_SUB, ROWS_PER_SUB)])
    plsc.subcore_barrier()

    def idxload(i, b):
        pltpu.async_copy(src_hbm.at[wid, i], si_v.at[b], isem.at[b])

    def idx_wait(i, b):
        pltpu.make_async_copy(src_hbm.at[wid, i], si_v.at[b], isem.at[b]).wait()

    def gather(b):
        pltpu.async_copy(table_hbm.at[si_v.at[b]], rows_v.at[b], gsem.at[b])

    def gather_wait(b):
        pltpu.make_async_copy(
            table_hbm.at[si_v.at[b]], rows_v.at[b], gsem.at[b]
        ).wait()

    def scatter(i, b):
        pltpu.async_copy(rows_v.at[b], acc_sh.at[dst_all.at[i]], ssem.at[b],
                         add=True)

    def scatter_wait(i, b):
        pltpu.make_async_copy(
            rows_v.at[b], acc_sh.at[dst_all.at[i]], ssem.at[b]
        ).wait()

    idxload(0, 0)
    idxload(1, 1)
    idx_wait(0, 0)
    gather(0)

    def body(g, carry):
        for b in range(2):
            i2 = g * 2 + b
            nb = 1 - b

            @pl.when(i2 >= 1)
            def _():
                scatter_wait(i2 - 1, nb)

            @pl.when(i2 + 1 < nch)
            def _():
                idx_wait(i2 + 1, nb)
                gather(nb)

            gather_wait(b)
            scatter(i2, b)

            @pl.when(i2 + 2 < nch)
            def _():
                idxload(i2 + 2, b)
        return carry

    lax.fori_loop(0, nch2, body, 0)
    scatter_wait(nch - 1, 1)

    plsc.subcore_barrier()

    pltpu.sync_copy(
        acc_sh.at[pl.ds(sid * ROWS_PER_SUB, ROWS_PER_SUB)],
        out_hbm.at[cid, pl.ds(sid * ROWS_PER_SUB, ROWS_PER_SUB)],
    )


def _tc_prescale_body(x_ref, w_ref, degp_ref, s_ref, d2_ref, dinv_ref):
    deg = jnp.sum(degp_ref[0, 0:N, :] + degp_ref[1, 0:N, :], axis=1,
                  keepdims=True) + 1.0
    dinv = lax.rsqrt(deg)
    xw = jnp.dot(x_ref[...], w_ref[...], preferred_element_type=jnp.float32)
    s = dinv * xw
    s_ref[...] = s
    d2_ref[...] = dinv * s
    dinv_ref[...] = dinv


_tc_prescale = pl.pallas_call(
    _tc_prescale_body,
    out_shape=[
        jax.ShapeDtypeStruct((N, H), jnp.float32),
        jax.ShapeDtypeStruct((N, H), jnp.float32),
        jax.ShapeDtypeStruct((N, 1), jnp.float32),
    ],
)


def _tc_mid_body(p_ref, d2_ref, dinv_ref, b1_ref, w2_ref, s_ref, d2o_ref):
    agg = p_ref[0, 0:N, :] + p_ref[1, 0:N, :]
    dinv = dinv_ref[...]
    h1 = jnp.maximum(dinv * agg + d2_ref[...] + b1_ref[...], 0.0)
    xw = jnp.dot(h1, w2_ref[...], preferred_element_type=jnp.float32)
    s = dinv * xw
    s_ref[...] = s
    d2o_ref[...] = dinv * s


_tc_mid = pl.pallas_call(
    _tc_mid_body,
    out_shape=[
        jax.ShapeDtypeStruct((N, H), jnp.float32),
        jax.ShapeDtypeStruct((N, H), jnp.float32),
    ],
)


def _tc_head_body(q_ref, d2_ref, dinv_ref, b2_ref, batch_ref, topo_ref,
                  wl_ref, bl_ref, out_ref):
    agg = q_ref[0, 0:N, :] + q_ref[1, 0:N, :]
    h2 = jnp.maximum(dinv_ref[...] * agg + d2_ref[...] + b2_ref[...], 0.0)
    gid = lax.broadcasted_iota(jnp.int32, (N, G), 1)
    onehot = jnp.where(gid == batch_ref[...], 1.0, 0.0)
    ssum = lax.dot_general(
        onehot, h2, (((0,), (0,)), ((), ())), preferred_element_type=jnp.float32
    )
    cnt = lax.dot_general(
        onehot, jnp.ones((N, 1), jnp.float32), (((0,), (0,)), ((), ())),
        preferred_element_type=jnp.float32,
    )
    g = jnp.where(cnt > 0.0, ssum / jnp.maximum(cnt, 1.0), 0.0)
    gt = jnp.concatenate([g, topo_ref[...]], axis=1)
    out_ref[...] = (
        jnp.dot(gt, wl_ref[...], preferred_element_type=jnp.float32) + bl_ref[...]
    )


_tc_head = pl.pallas_call(
    _tc_head_body,
    out_shape=jax.ShapeDtypeStruct((G, C), jnp.float32),
)


def kernel(x, edge_index, batch, topo, W1, b1, W2, b2, Wl, bl):
    src = edge_index[0]
    dst = edge_index[1]
    pad = EPAD - E
    iota_pad = jnp.arange(pad, dtype=jnp.int32)
    src_p = jnp.concatenate([src, jnp.zeros((pad,), jnp.int32)])
    dst_msg = jnp.concatenate([dst, N + iota_pad % (NACC - N)])
    dst_deg = jnp.concatenate([dst, N + iota_pad % (NPAD - N)])
    dst2d = dst_deg.reshape(NC, NS * EPT)

    def to_tiles(e):
        cut = NS * CA * CHUNK
        fastp = e[:cut].reshape(NS, CA, CHUNK)
        slowp = jnp.pad(e[cut:].reshape(NS, CB, CHUNK),
                        ((0, 0), (0, CMAX - CB), (0, 0)))
        parts = [fastp, slowp] if FAST_CORE == 0 else [slowp, fastp]
        return jnp.concatenate(parts, axis=0)

    src3 = to_tiles(src_p)
    dst3m = to_tiles(dst_msg)

    zrow = jnp.zeros((ROWS_PER_SUB, H), jnp.float32)

    degp = _sc_degree(dst2d)
    s1, d2xw1, dinv = _tc_prescale(x, W1, degp)
    p = _sc_spmm(s1, src3, dst3m, zrow)
    s2, d2xw2 = _tc_mid(p, d2xw1, dinv, b1.reshape(1, H), W2)
    q = _sc_spmm(s2, src3, dst3m, zrow)
    out = _tc_head(
        q, d2xw2, dinv, b2.reshape(1, H), batch.reshape(N, 1), topo,
        Wl, bl.reshape(1, C),
    )
    return out
